# Initial kernel scaffold; baseline (speedup 1.0000x reference)
import jax
import jax.numpy as jnp
from jax.experimental import pallas as pl


def kernel(x, edge_index, batch, W_enc, b_enc, W_r1, b_r1, W_r2, b_r2, We0_root, We0_nbr, be0, We1_root, We1_nbr, be1):
    raise NotImplementedError("write your pallas kernel here")


# speedup vs baseline: 5.7758x; 5.7758x over previous
import functools

import jax
import jax.numpy as jnp
from jax import lax
from jax.experimental import pallas as pl
from jax.experimental.pallas import tpu as pltpu
from jax.experimental.pallas import tpu_sc as plsc

_N = 10000
_E = 160000
_IN = 256
_H = 512
_OUT = 256
_NE = 8
_NG = 64
_BN = 1000
_NB = _N // _BN

_PREC = None

_pcall = pl.pallas_call


def _encrouter_body(x_ref, we_ref, be_ref, nc_ref, ec_ref, wr1_ref,
                    br1_ref, wr2_ref, br2_ref, batch_ref,
                    h_ref, idx_ref, hist_ref):
    i = pl.program_id(0)
    x = x_ref[...]
    h = jnp.maximum(jnp.dot(x, we_ref[...], precision=_PREC) + be_ref[0, :], 0.0)
    h_ref[...] = h
    sfn = jnp.log1p(nc_ref[0, :])
    sfe = jnp.log1p(ec_ref[0, :])
    sf_tab = jnp.stack([sfn, sfe], axis=-1)
    b = batch_ref[0, 0, :]
    onehot = (b[:, None] == lax.broadcasted_iota(jnp.int32, (1, _NG), 1)
              ).astype(jnp.float32)
    sf = jnp.dot(onehot, sf_tab, precision=lax.Precision.HIGHEST)
    rin = jnp.concatenate([h, sf], axis=1)
    r1 = jnp.maximum(jnp.dot(rin, wr1_ref[...], precision=_PREC)
                     + br1_ref[0, :], 0.0)
    logits = jnp.dot(r1, wr2_ref[...], precision=_PREC) + br2_ref[0, :]
    m = jnp.max(logits, axis=-1, keepdims=True)
    ids = lax.broadcasted_iota(jnp.int32, (_BN, _NE), 1)
    eidx = jnp.min(jnp.where(logits >= m, ids, _NE), axis=-1)
    idx_ref[0, 0, :] = eidx
    oh8 = (eidx[:, None] == lax.broadcasted_iota(jnp.int32, (1, _NE), 1)
           ).astype(jnp.int32)

    @pl.when(i == 0)
    def _():
        hist_ref[...] = jnp.zeros_like(hist_ref)

    hist_ref[0, :] += jnp.sum(oh8, axis=0)


def _encrouter(x, w_enc, b_enc, ncount, ecount, w_r1, b_r1, w_r2, b_r2,
               batch3):
    return _pcall(
        _encrouter_body,
        grid=(_NB,),
        in_specs=[
            pl.BlockSpec((_BN, _IN), lambda i: (i, 0)),
            pl.BlockSpec((_IN, _H), lambda i: (0, 0)),
            pl.BlockSpec((1, _H), lambda i: (0, 0)),
            pl.BlockSpec((1, _NG), lambda i: (0, 0)),
            pl.BlockSpec((1, _NG), lambda i: (0, 0)),
            pl.BlockSpec((_H + 2, _H), lambda i: (0, 0)),
            pl.BlockSpec((1, _H), lambda i: (0, 0)),
            pl.BlockSpec((_H, _NE), lambda i: (0, 0)),
            pl.BlockSpec((1, _NE), lambda i: (0, 0)),
            pl.BlockSpec((1, 1, _BN), lambda i: (i, 0, 0)),
        ],
        out_specs=[
            pl.BlockSpec((_BN, _H), lambda i: (i, 0)),
            pl.BlockSpec((1, 1, _BN), lambda i: (i, 0, 0)),
            pl.BlockSpec((1, _NE), lambda i: (0, 0)),
        ],
        out_shape=[
            jax.ShapeDtypeStruct((_N, _H), jnp.float32),
            jax.ShapeDtypeStruct((_NB, 1, _BN), jnp.int32),
            jax.ShapeDtypeStruct((1, _NE), jnp.int32),
        ],
    )(x, w_enc, b_enc, ncount, ecount, w_r1, b_r1, w_r2, b_r2, batch3)


def _layer1_body(h_ref, agg0_ref, wr_ref, wn_ref, b_ref, h1_ref):
    h1 = (jnp.dot(h_ref[...], wr_ref[0], precision=_PREC)
          + jnp.dot(agg0_ref[...], wn_ref[0], precision=_PREC) + b_ref[0, 0, :])
    h1_ref[...] = jnp.maximum(h1, 0.0)[None, :, :]


def _layer1(h, agg0, we0_root, we0_nbr, be0_3):
    return _pcall(
        _layer1_body,
        grid=(_NB, _NE),
        in_specs=[
            pl.BlockSpec((_BN, _H), lambda i, e: (i, 0)),
            pl.BlockSpec((_BN, _H), lambda i, e: (i, 0)),
            pl.BlockSpec((1, _H, _H), lambda i, e: (e, 0, 0)),
            pl.BlockSpec((1, _H, _H), lambda i, e: (e, 0, 0)),
            pl.BlockSpec((1, 1, _H), lambda i, e: (e, 0, 0)),
        ],
        out_specs=pl.BlockSpec((1, _BN, _H), lambda i, e: (e, i, 0)),
        out_shape=jax.ShapeDtypeStruct((_NE, _N, _H), jnp.float32),
    )(h, agg0, we0_root, we0_nbr, be0_3)


def _layer2_body(h1_ref, agg1_ref, wr_ref, wn_ref, b_ref, idx_ref, out_ref):
    e = pl.program_id(1)
    sel = (idx_ref[0, 0, :] == e).astype(jnp.float32)
    y = (jnp.dot(h1_ref[0], wr_ref[0], precision=_PREC)
         + jnp.dot(agg1_ref[...], wn_ref[0], precision=_PREC) + b_ref[0, 0, :])
    contrib = y * sel[:, None]

    @pl.when(e == 0)
    def _():
        out_ref[...] = jnp.zeros_like(out_ref)

    out_ref[...] += contrib


def _layer2(h1, agg1, we1_root, we1_nbr, be1_3, idx3):
    return _pcall(
        _layer2_body,
        grid=(_NB, _NE),
        in_specs=[
            pl.BlockSpec((1, _BN, _H), lambda i, e: (e, i, 0)),
            pl.BlockSpec((_BN, _H), lambda i, e: (i, 0)),
            pl.BlockSpec((1, _H, _OUT), lambda i, e: (e, 0, 0)),
            pl.BlockSpec((1, _H, _OUT), lambda i, e: (e, 0, 0)),
            pl.BlockSpec((1, 1, _OUT), lambda i, e: (e, 0, 0)),
            pl.BlockSpec((1, 1, _BN), lambda i, e: (i, 0, 0)),
        ],
        out_specs=pl.BlockSpec((_BN, _OUT), lambda i, e: (i, 0)),
        out_shape=jax.ShapeDtypeStruct((_N, _OUT), jnp.float32),
    )(h1, agg1, we1_root, we1_nbr, be1_3, idx3)


def kernel(x, edge_index, batch, W_enc, b_enc, W_r1, b_r1, W_r2, b_r2,
           We0_root, We0_nbr, be0, We1_root, We1_nbr, be1):
    src = edge_index[0]
    dst = edge_index[1]
    batch3 = batch.astype(jnp.int32).reshape(_NB, 1, _BN)

    ncount = jnp.bincount(batch, length=_NG).astype(jnp.float32).reshape(1, _NG)
    ecount = jnp.bincount(batch[src], length=_NG).astype(jnp.float32).reshape(1, _NG)

    h, idx3, hist = _encrouter(
        x, W_enc, b_enc.reshape(1, _H), ncount, ecount,
        W_r1, b_r1.reshape(1, _H), W_r2, b_r2.reshape(1, _NE), batch3)
    idx = idx3.reshape(_N)

    agg0 = jax.ops.segment_sum(h[src], dst, num_segments=_N)

    h1 = _layer1(h, agg0, We0_root, We0_nbr, be0.reshape(_NE, 1, _H))

    rows = idx[dst] * _N + src
    agg1 = jax.ops.segment_sum(h1.reshape(_NE * _N, _H)[rows], dst,
                               num_segments=_N)

    out = _layer2(h1, agg1, We1_root, We1_nbr, be1.reshape(_NE, 1, _OUT), idx3)
    return out, hist.reshape(_NE)


# speedup vs baseline: 8.0851x; 1.3998x over previous
import functools

import jax
import jax.numpy as jnp
from jax import lax
from jax.experimental import pallas as pl
from jax.experimental.pallas import tpu as pltpu
from jax.experimental.pallas import tpu_sc as plsc

_N = 10000
_E = 160000
_IN = 256
_H = 512
_OUT = 256
_NE = 8
_NG = 64
_BN = 1000
_NB = _N // _BN

_PREC = None

_pcall = pl.pallas_call


def _encrouter_body(x_ref, we_ref, be_ref, nc_ref, ec_ref, wr1_ref,
                    br1_ref, wr2_ref, br2_ref, batch_ref,
                    h_ref, idx_ref, hist_ref):
    i = pl.program_id(0)
    x = x_ref[...]
    h = jnp.maximum(jnp.dot(x, we_ref[...], precision=_PREC) + be_ref[0, :], 0.0)
    h_ref[...] = h
    sfn = jnp.log1p(nc_ref[0, :])
    sfe = jnp.log1p(ec_ref[0, :])
    sf_tab = jnp.stack([sfn, sfe], axis=-1)
    b = batch_ref[0, 0, :]
    onehot = (b[:, None] == lax.broadcasted_iota(jnp.int32, (1, _NG), 1)
              ).astype(jnp.float32)
    sf = jnp.dot(onehot, sf_tab, precision=lax.Precision.HIGHEST)
    rin = jnp.concatenate([h, sf], axis=1)
    r1 = jnp.maximum(jnp.dot(rin, wr1_ref[...], precision=_PREC)
                     + br1_ref[0, :], 0.0)
    logits = jnp.dot(r1, wr2_ref[...], precision=_PREC) + br2_ref[0, :]
    m = jnp.max(logits, axis=-1, keepdims=True)
    ids = lax.broadcasted_iota(jnp.int32, (_BN, _NE), 1)
    eidx = jnp.min(jnp.where(logits >= m, ids, _NE), axis=-1)
    idx_ref[0, 0, :] = eidx
    oh8 = (eidx[:, None] == lax.broadcasted_iota(jnp.int32, (1, _NE), 1)
           ).astype(jnp.int32)

    @pl.when(i == 0)
    def _():
        hist_ref[...] = jnp.zeros_like(hist_ref)

    hist_ref[0, :] += jnp.sum(oh8, axis=0)


def _encrouter(x, w_enc, b_enc, ncount, ecount, w_r1, b_r1, w_r2, b_r2,
               batch3):
    return _pcall(
        _encrouter_body,
        grid=(_NB,),
        in_specs=[
            pl.BlockSpec((_BN, _IN), lambda i: (i, 0)),
            pl.BlockSpec((_IN, _H), lambda i: (0, 0)),
            pl.BlockSpec((1, _H), lambda i: (0, 0)),
            pl.BlockSpec((1, _NG), lambda i: (0, 0)),
            pl.BlockSpec((1, _NG), lambda i: (0, 0)),
            pl.BlockSpec((_H + 2, _H), lambda i: (0, 0)),
            pl.BlockSpec((1, _H), lambda i: (0, 0)),
            pl.BlockSpec((_H, _NE), lambda i: (0, 0)),
            pl.BlockSpec((1, _NE), lambda i: (0, 0)),
            pl.BlockSpec((1, 1, _BN), lambda i: (i, 0, 0)),
        ],
        out_specs=[
            pl.BlockSpec((_BN, _H), lambda i: (i, 0)),
            pl.BlockSpec((1, 1, _BN), lambda i: (i, 0, 0)),
            pl.BlockSpec((1, _NE), lambda i: (0, 0)),
        ],
        out_shape=[
            jax.ShapeDtypeStruct((_N, _H), jnp.float32),
            jax.ShapeDtypeStruct((_NB, 1, _BN), jnp.int32),
            jax.ShapeDtypeStruct((1, _NE), jnp.int32),
        ],
    )(x, w_enc, b_enc, ncount, ecount, w_r1, b_r1, w_r2, b_r2, batch3)


def _layer1_body(h_ref, agg0_ref, wr_ref, wn_ref, b_ref, h1_ref):
    h1 = (jnp.dot(h_ref[...], wr_ref[0], precision=_PREC)
          + jnp.dot(agg0_ref[...], wn_ref[0], precision=_PREC) + b_ref[0, 0, :])
    h1_ref[...] = jnp.maximum(h1, 0.0)[None, :, :]


def _layer1(h, agg0, we0_root, we0_nbr, be0_3):
    return _pcall(
        _layer1_body,
        grid=(_NB, _NE),
        in_specs=[
            pl.BlockSpec((_BN, _H), lambda i, e: (i, 0)),
            pl.BlockSpec((_BN, _H), lambda i, e: (i, 0)),
            pl.BlockSpec((1, _H, _H), lambda i, e: (e, 0, 0)),
            pl.BlockSpec((1, _H, _H), lambda i, e: (e, 0, 0)),
            pl.BlockSpec((1, 1, _H), lambda i, e: (e, 0, 0)),
        ],
        out_specs=pl.BlockSpec((1, _BN, _H), lambda i, e: (e, i, 0)),
        out_shape=jax.ShapeDtypeStruct((_NE, _N, _H), jnp.float32),
    )(h, agg0, we0_root, we0_nbr, be0_3)


def _layer2_body(h1_ref, agg1_ref, wr_ref, wn_ref, b_ref, idx_ref, out_ref):
    e = pl.program_id(1)
    sel = (idx_ref[0, 0, :] == e).astype(jnp.float32)
    y = (jnp.dot(h1_ref[0], wr_ref[0], precision=_PREC)
         + jnp.dot(agg1_ref[...], wn_ref[0], precision=_PREC) + b_ref[0, 0, :])
    contrib = y * sel[:, None]

    @pl.when(e == 0)
    def _():
        out_ref[...] = jnp.zeros_like(out_ref)

    out_ref[...] += contrib


def _layer2(h1, agg1, we1_root, we1_nbr, be1_3, idx3):
    return _pcall(
        _layer2_body,
        grid=(_NB, _NE),
        in_specs=[
            pl.BlockSpec((1, _BN, _H), lambda i, e: (e, i, 0)),
            pl.BlockSpec((_BN, _H), lambda i, e: (i, 0)),
            pl.BlockSpec((1, _H, _OUT), lambda i, e: (e, 0, 0)),
            pl.BlockSpec((1, _H, _OUT), lambda i, e: (e, 0, 0)),
            pl.BlockSpec((1, 1, _OUT), lambda i, e: (e, 0, 0)),
            pl.BlockSpec((1, 1, _BN), lambda i, e: (i, 0, 0)),
        ],
        out_specs=pl.BlockSpec((_BN, _OUT), lambda i, e: (i, 0)),
        out_shape=jax.ShapeDtypeStruct((_N, _OUT), jnp.float32),
    )(h1, agg1, we1_root, we1_nbr, be1_3, idx3)


_NC = 2
_NS = 16
_L = 16
_NW = _NC * _NS
_EPAD = 160256
_CE = _EPAD // _NW
_NPAD = 10240
_SR = 160
_G = 16
_MAXCH = (_CE + _G - 1) // _G + 1


def _counts_body(batchp, srcp, outn, oute, btab, sbuf, accn, acce, obuf):
    c = lax.axis_index("c")
    s = lax.axis_index("s")
    wid = c * _NS + s
    lanes = lax.iota(jnp.int32, 16)
    ones = jnp.ones((16,), jnp.float32)
    pltpu.sync_copy(batchp, btab)
    pltpu.sync_copy(srcp.at[pl.ds(wid * _CE, _CE)], sbuf)

    def zb(j, _):
        accn[pl.ds(j * 16, 16)] = jnp.zeros((16,), jnp.float32)
        acce[pl.ds(j * 16, 16)] = jnp.zeros((16,), jnp.float32)
        return 0

    lax.fori_loop(0, 64, zb, 0)

    def nb(j, _):
        off = wid * 320 + j * 16
        g = btab[pl.ds(off, 16)]
        m = (off + lanes) < _N
        a = lanes * _NG + g
        v = plsc.load_gather(accn, [a])
        plsc.store_scatter(accn, [a], v + ones, mask=m)
        return 0

    lax.fori_loop(0, 20, nb, 0)

    def eb(j, _):
        sv = sbuf[pl.ds(j * 16, 16)]
        m = (wid * _CE + j * 16 + lanes) < _E
        gv = plsc.load_gather(btab, [sv])
        a = lanes * _NG + gv
        v = plsc.load_gather(acce, [a])
        plsc.store_scatter(acce, [a], v + ones, mask=m)
        return 0

    lax.fori_loop(0, _CE // 16, eb, 0)

    for acc, out in ((accn, outn), (acce, oute)):
        for k in range(_NG // 16):
            tot = jnp.zeros((16,), jnp.float32)
            for l in range(16):
                tot = tot + acc[pl.ds(l * _NG + k * 16, 16)]
            obuf[pl.ds(k * 16, 16)] = tot
        pltpu.sync_copy(obuf, out.at[wid])


def _sc_counts(batchp, srcp):
    mesh = plsc.VectorSubcoreMesh(core_axis_name="c", subcore_axis_name="s")
    f = pl.kernel(
        _counts_body,
        compiler_params=pltpu.CompilerParams(needs_layout_passes=False),
        out_type=[jax.ShapeDtypeStruct((_NW, _NG), jnp.float32),
                  jax.ShapeDtypeStruct((_NW, _NG), jnp.float32)],
        mesh=mesh,
        scratch_types=[
            pltpu.VMEM((_NPAD,), jnp.int32),
            pltpu.VMEM((_CE,), jnp.int32),
            pltpu.VMEM((16 * _NG,), jnp.float32),
            pltpu.VMEM((16 * _NG,), jnp.float32),
            pltpu.VMEM((_NG,), jnp.float32),
        ],
    )
    return f(batchp, srcp)


def _make_segsum_body(nrows, with_sel):
    def body(table, srcp, dstp, *rest):
        if with_sel:
            (selp, out, srcb, dstb, stab, rowf, dlf, stage, acc, sem) = rest
        else:
            (out, srcb, dstb, rowf, dlf, stage, acc, sem) = rest
            selp = stab = None
        c = lax.axis_index("c")
        s = lax.axis_index("s")
        wid = c * _NS + s
        lanes = lax.iota(jnp.int32, 16)
        if with_sel:
            pltpu.sync_copy(selp, stab)

        for p in range(2):
            base = p * (_NW * _SR) + wid * _SR

            def zz(r, _):
                for k in range(_H // 16):
                    acc[r, pl.ds(k * 16, 16)] = jnp.zeros((16,), jnp.float32)
                return 0

            lax.fori_loop(0, _SR, zz, 0)

            def blk_body(blk, _):
                b0 = pl.multiple_of(blk * _CE, 8)
                pltpu.sync_copy(dstp.at[pl.ds(b0, _CE)], dstb)
                pltpu.sync_copy(srcp.at[pl.ds(b0, _CE)], srcb)

                def fb(j, off):
                    d = dstb[pl.ds(j * 16, 16)]
                    sv = srcb[pl.ds(j * 16, 16)]
                    valid = (blk * _CE + j * 16 + lanes) < _E
                    m = valid & (d >= base) & (d < base + _SR)
                    if with_sel:
                        sel = plsc.load_gather(stab, [d])
                        rowv = sel * _N + sv
                    else:
                        rowv = sv
                    plsc.store_compressed(rowf.at[pl.ds(off, 16)], rowv,
                                          mask=m)
                    plsc.store_compressed(dlf.at[pl.ds(off, 16)], d - base,
                                          mask=m)
                    cnt = jnp.max(plsc.all_reduce_population_count(m))
                    return off + cnt

                m_cnt = lax.fori_loop(0, _CE // 16, fb, jnp.int32(0))
                rowf[pl.ds(m_cnt, 16)] = jnp.zeros((16,), jnp.int32)
                dlf[pl.ds(m_cnt, 16)] = jnp.full((16,), _SR, jnp.int32)
                nch = (m_cnt + _G - 1) // _G

                def mv(k, _):
                    pltpu.async_copy(table.at[rowf.at[pl.ds(k * _G, _G)]],
                                     stage, sem).wait()
                    dlv = dlf[pl.ds(k * _G, 16)]
                    for j in range(_G):
                        dl = dlv[j]
                        for kk in range(_H // 16):
                            plsc.addupdate(acc.at[dl, pl.ds(kk * 16, 16)],
                                           stage[j, pl.ds(kk * 16, 16)])
                    return 0

                lax.fori_loop(0, nch, mv, 0)
                return 0

            lax.fori_loop(0, _EPAD // _CE, blk_body, 0)

            o0 = pl.multiple_of(base, 8)
            pltpu.sync_copy(acc.at[pl.ds(0, _SR)], out.at[pl.ds(o0, _SR)])

    return body


def _sc_segsum(table, srcp, dstp, selp=None):
    with_sel = selp is not None
    mesh = plsc.VectorSubcoreMesh(core_axis_name="c", subcore_axis_name="s")
    scratch = [
        pltpu.VMEM((_CE,), jnp.int32),
        pltpu.VMEM((_CE,), jnp.int32),
    ]
    if with_sel:
        scratch.append(pltpu.VMEM((_NPAD,), jnp.int32))
    scratch += [
        pltpu.VMEM((_CE + 2 * _L,), jnp.int32),
        pltpu.VMEM((_CE + 2 * _L,), jnp.int32),
        pltpu.VMEM((_G, _H), jnp.float32),
        pltpu.VMEM((_SR + 8, _H), jnp.float32),
        pltpu.SemaphoreType.DMA,
    ]
    f = pl.kernel(
        _make_segsum_body(table.shape[0], with_sel),
        compiler_params=pltpu.CompilerParams(needs_layout_passes=False),
        out_type=jax.ShapeDtypeStruct((_NPAD, _H), jnp.float32),
        mesh=mesh,
        scratch_types=scratch,
    )
    args = (table, srcp, dstp) + ((selp,) if with_sel else ())
    return f(*args)


def kernel(x, edge_index, batch, W_enc, b_enc, W_r1, b_r1, W_r2, b_r2,
           We0_root, We0_nbr, be0, We1_root, We1_nbr, be1):
    src = edge_index[0].astype(jnp.int32)
    dst = edge_index[1].astype(jnp.int32)
    batch = batch.astype(jnp.int32)
    batch3 = batch.reshape(_NB, 1, _BN)
    srcp = jnp.zeros((_EPAD,), jnp.int32).at[:_E].set(src)
    dstp = jnp.zeros((_EPAD,), jnp.int32).at[:_E].set(dst)
    batchp = jnp.zeros((_NPAD,), jnp.int32).at[:_N].set(batch)

    npart, epart = _sc_counts(batchp, srcp)
    ncount = jnp.sum(npart, axis=0).reshape(1, _NG)
    ecount = jnp.sum(epart, axis=0).reshape(1, _NG)

    h, idx3, hist = _encrouter(
        x, W_enc, b_enc.reshape(1, _H), ncount, ecount,
        W_r1, b_r1.reshape(1, _H), W_r2, b_r2.reshape(1, _NE), batch3)
    idx = idx3.reshape(_N)

    agg0 = _sc_segsum(h, srcp, dstp)[:_N]

    h1 = _layer1(h, agg0, We0_root, We0_nbr, be0.reshape(_NE, 1, _H))

    idxp = jnp.zeros((_NPAD,), jnp.int32).at[:_N].set(idx)
    agg1 = _sc_segsum(h1.reshape(_NE * _N, _H), srcp, dstp, idxp)[:_N]

    out = _layer2(h1, agg1, We1_root, We1_nbr, be1.reshape(_NE, 1, _OUT), idx3)
    return out, hist.reshape(_NE)
